# baseline (speedup 1.0000x reference)
import functools

import jax
import jax.numpy as jnp
from jax import lax
from jax.experimental import pallas as pl
from jax.experimental.pallas import tpu as pltpu
from jax.experimental.pallas import tpu_sc as plsc

CLS = 100000
FEAT = 64
BATCH_N = 16384

_NC = 2
_NS = 16
_NW = _NC * _NS
_BPW = BATCH_N // _NW
_IDX_CHUNK = 128
_NCHUNK = _BPW // _IDX_CHUNK
_L = 16
_GROUPS = FEAT // _L


@functools.partial(
    pl.kernel,
    out_type=jax.ShapeDtypeStruct((_NW, _L), jnp.float32),
    mesh=plsc.VectorSubcoreMesh(
        core_axis_name="c", subcore_axis_name="s",
        num_cores=_NC, num_subcores=_NS,
    ),
    scratch_types=[
        pltpu.VMEM((_NCHUNK, _IDX_CHUNK), jnp.int32),
        pltpu.VMEM((_BPW, FEAT), jnp.float32),
        pltpu.VMEM((_BPW, FEAT), jnp.float32),
        pltpu.VMEM((_L,), jnp.float32),
        pltpu.SemaphoreType.DMA,
        pltpu.SemaphoreType.DMA,
    ],
    compiler_params=pltpu.CompilerParams(use_tc_tiling_on_sc=False),
)
def _center_partials(xs_hbm, label_hbm, center_hbm, out_hbm,
                     idx_v, rows_v, xs_v, acc_v, gsem, xsem):
    wid = lax.axis_index("s") * _NC + lax.axis_index("c")
    base = wid * _BPW

    xs_cp = pltpu.async_copy(xs_hbm.at[pl.ds(base, _BPW), :], xs_v, xsem)
    for c in range(_NCHUNK):
        pltpu.sync_copy(
            label_hbm.at[pl.ds(base + c * _IDX_CHUNK, _IDX_CHUNK)],
            idx_v.at[c])
    gathers = [
        pltpu.async_copy(
            center_hbm.at[idx_v.at[c]],
            rows_v.at[pl.ds(c * _IDX_CHUNK, _IDX_CHUNK), :],
            gsem)
        for c in range(_NCHUNK)
    ]
    xs_cp.wait()
    for g in gathers:
        g.wait()

    zeros = jnp.zeros((_L,), jnp.float32)

    def body(r, accs):
        new = []
        for g in range(_GROUPS):
            d = xs_v[r, pl.ds(g * _L, _L)] - rows_v[r, pl.ds(g * _L, _L)]
            new.append(accs[g] + d * d)
        return tuple(new)

    accs = lax.fori_loop(0, _BPW, body, (zeros,) * _GROUPS)
    acc_v[...] = accs[0] + accs[1] + accs[2] + accs[3]
    pltpu.sync_copy(acc_v, out_hbm.at[wid])


def _tc_reduce_body(p_ref, o_ref):
    o_ref[...] = (jnp.sum(p_ref[...]) * (2.0 / BATCH_N))[None, None]


def kernel(xs, label, center):
    partials = _center_partials(xs, label.astype(jnp.int32), center)
    loss = pl.pallas_call(
        _tc_reduce_body,
        out_shape=jax.ShapeDtypeStruct((1, 1), jnp.float32),
    )(partials)
    return loss.reshape((1,))


# speedup vs baseline: 2.2426x; 2.2426x over previous
import functools

import jax
import jax.numpy as jnp
from jax import lax
from jax.experimental import pallas as pl
from jax.experimental.pallas import tpu as pltpu
from jax.experimental.pallas import tpu_sc as plsc

CLS = 100000
FEAT = 64
BATCH_N = 16384

_NC = 2
_NS = 16
_NW = _NC * _NS
_FPW = FEAT // _NW
_L = 16
_HALF = BATCH_N // 2
_GRP = 4
_ITERS = _HALF // (_L * _GRP)


@functools.partial(
    pl.kernel,
    out_type=jax.ShapeDtypeStruct((_NW * _L,), jnp.float32),
    mesh=plsc.VectorSubcoreMesh(
        core_axis_name="c", subcore_axis_name="s",
        num_cores=_NC, num_subcores=_NS,
    ),
    scratch_types=[
        pltpu.VMEM((CLS,), jnp.float32),
        pltpu.VMEM((BATCH_N,), jnp.int32),
        pltpu.VMEM((_HALF,), jnp.float32),
        pltpu.VMEM((_L,), jnp.float32),
        pltpu.SemaphoreType.DMA,
    ],
    compiler_params=pltpu.CompilerParams(needs_layout_passes=False),
)
def _center_partials(xs_t_hbm, label_hbm, center_t_hbm, out_hbm,
                     row_v, lab_v, xs_v, acc_v, sem):
    wid = lax.axis_index("s") * _NC + lax.axis_index("c")

    pltpu.sync_copy(label_hbm, lab_v)

    zeros = jnp.zeros((_L,), jnp.float32)
    accs = (zeros,) * _GRP
    for fi in range(_FPW):
        f = wid * _FPW + fi
        pltpu.sync_copy(center_t_hbm.at[f], row_v)
        for h in range(2):
            pltpu.sync_copy(xs_t_hbm.at[f, pl.ds(h * _HALF, _HALF)], xs_v)
            lab_base = h * _HALF

            def body(i, accs, lab_base=lab_base):
                out = []
                for g in range(_GRP):
                    o = i * (_L * _GRP) + g * _L
                    idx = lab_v[pl.ds(lab_base + o, _L)]
                    gathered = plsc.load_gather(row_v, [idx])
                    d = xs_v[pl.ds(o, _L)] - gathered
                    out.append(accs[g] + d * d)
                return tuple(out)

            accs = lax.fori_loop(0, _ITERS, body, accs)

    acc_v[...] = (accs[0] + accs[1]) + (accs[2] + accs[3])
    pltpu.sync_copy(acc_v, out_hbm.at[pl.ds(wid * _L, _L)])


def _tc_reduce_body(p_ref, o_ref):
    o_ref[...] = (jnp.sum(p_ref[...]) * (2.0 / BATCH_N))[None, None]


def kernel(xs, label, center):
    partials = _center_partials(xs.T, label.astype(jnp.int32), center.T)
    loss = pl.pallas_call(
        _tc_reduce_body,
        out_shape=jax.ShapeDtypeStruct((1, 1), jnp.float32),
    )(partials)
    return loss.reshape((1,))
